# Initial kernel scaffold; baseline (speedup 1.0000x reference)
import jax
import jax.numpy as jnp
from jax.experimental import pallas as pl


def kernel(edge_index, edge_weight, home, away, emb, beta1, beta2, W1, b1, W2, b2, W3, b3):
    raise NotImplementedError("write your pallas kernel here")


# speedup vs baseline: 40.2064x; 40.2064x over previous
import jax
import jax.numpy as jnp
from jax import lax
from jax.experimental import pallas as pl
from jax.experimental.pallas import tpu as pltpu
from jax.experimental.pallas import tpu_sc as plsc

_N = 100000
_E = 1600000
_D = 16
_B = 16384
_TD = 3

_NC, _NS, _L = 2, 16, 16
_NW = _NC * _NS
_NP = 100352
_CH = 128
_TCH = _E // _CH
_RPT = _NP // _NS

_f32 = jnp.float32
_i32 = jnp.int32


_ZNC = 49
_ZR = _RPT // _ZNC
_DZR = _RPT // 8
_TRIPS = -(-_TCH // _NW + 5) // 6 * 6


def _tree_sum(vs):
    while len(vs) > 1:
        nxt = [vs[i] + vs[i + 1] for i in range(0, len(vs) - 1, 2)]
        if len(vs) % 2:
            nxt.append(vs[-1])
        vs = nxt
    return vs[0]


def _edge_body(xn, nrm, ei, ew, beta16,
               acc_out, den_out,
               acc_sh, den_sh, idxs, ws, nss, num_v,
               xss, xds, rows_v, beta_v, zb_v, zd_v,
               sidx, sg):
    c = lax.axis_index("c")
    s = lax.axis_index("s")
    wid = s * _NC + c
    nb = s * _RPT

    z16 = jnp.zeros((_L,), _f32)

    def zrow(i, carry):
        zb_v[i] = z16
        return carry

    lax.fori_loop(0, _ZR, zrow, 0)

    def zden(i, carry):
        zd_v[pl.ds(i * _L, _L)] = z16
        return carry

    lax.fori_loop(0, _DZR // _L, zden, 0)
    for k in range(_ZNC):
        pltpu.sync_copy(zb_v, acc_sh.at[pl.ds(nb + k * _ZR, _ZR)])
    for k in range(8):
        pltpu.sync_copy(zd_v, den_sh.at[pl.ds(nb + k * _DZR, _DZR)])
    pltpu.sync_copy(beta16, beta_v)
    plsc.subcore_barrier()

    bvec = beta_v[...]
    rows0 = lax.iota(_i32, _L)

    def ebase(i):
        return jnp.minimum(wid + i * _NW, _TCH - 1) * _CH

    def issue_idx(i, p):
        eb = ebase(i)
        pltpu.async_copy(ei.at[:, pl.ds(eb, _CH)], idxs[p].at[...], sidx[p])
        pltpu.async_copy(ew.at[pl.ds(eb, _CH)], ws[p].at[...], sidx[p])

    def wait_idx(i, p):
        eb = ebase(i)
        pltpu.make_async_copy(ei.at[:, pl.ds(eb, _CH)], idxs[p].at[...],
                              sidx[p]).wait()
        pltpu.make_async_copy(ew.at[pl.ds(eb, _CH)], ws[p].at[...],
                              sidx[p]).wait()

    def issue_gather(p, q):
        pltpu.async_copy(xn.at[idxs[p].at[0]], xss[q].at[...], sg[q])
        pltpu.async_copy(xn.at[idxs[p].at[1]], xds[q].at[...], sg[q])
        pltpu.async_copy(nrm.at[idxs[p].at[0]], nss[q].at[...], sg[q])

    def wait_gather(p, q):
        pltpu.make_async_copy(xn.at[idxs[p].at[0]], xss[q].at[...],
                              sg[q]).wait()
        pltpu.make_async_copy(xn.at[idxs[p].at[1]], xds[q].at[...],
                              sg[q]).wait()
        pltpu.make_async_copy(nrm.at[idxs[p].at[0]], nss[q].at[...],
                              sg[q]).wait()

    issue_idx(0, 0)
    issue_idx(1, 1)
    wait_idx(0, 0)
    issue_gather(0, 0)

    def compute(i, pi, pg):
        xs_v, xd_v, ns_v, w_v = xss[pg], xds[pg], nss[pg], ws[pi]
        vvalid = jnp.full((_L,), jnp.where(wid + i * _NW < _TCH, 1.0, 0.0),
                          _f32)
        for g in range(_CH // _L):
            rows = rows0 + g * _L
            sl = pl.ds(g * _L, _L)
            a_cols = []
            prods = []
            for d in range(_D):
                cd = jnp.full((_L,), d, _i32)
                a = plsc.load_gather(xs_v, [rows, cd])
                b = plsc.load_gather(xd_v, [rows, cd])
                a_cols.append(a)
                prods.append(a * b)
            dot = _tree_sum(prods)
            num = w_v[sl] * jnp.exp(bvec * dot) * vvalid
            num_v[sl] = num
            scale = num * ns_v[sl]
            for d in range(_D):
                cd = jnp.full((_L,), d, _i32)
                plsc.store_scatter(rows_v, [rows, cd], a_cols[d] * scale)
        pltpu.sync_copy(num_v, den_sh.at[idxs[pi].at[1]], add=True)
        pltpu.sync_copy(rows_v, acc_sh.at[idxs[pi].at[1]], add=True)

    def six(g6, carry):
        for b in range(6):
            i = g6 * 6 + b
            pi0, pi1, pi2 = b % 3, (b + 1) % 3, (b + 2) % 3
            pg0, pg1 = b % 2, (b + 1) % 2
            wait_idx(i + 1, pi1)
            issue_gather(pi1, pg1)
            issue_idx(i + 2, pi2)
            wait_gather(pi0, pg0)
            compute(i, pi0, pg0)
        return carry

    lax.fori_loop(0, _TRIPS // 6, six, 0)
    wait_gather(_TRIPS % 3, _TRIPS % 2)
    wait_idx(_TRIPS + 1, (_TRIPS + 1) % 3)
    plsc.subcore_barrier()
    for k in range(_ZNC):
        pltpu.sync_copy(acc_sh.at[pl.ds(nb + k * _ZR, _ZR)], zb_v)
        pltpu.sync_copy(zb_v, acc_out.at[c, pl.ds(nb + k * _ZR, _ZR)])
    for k in range(8):
        pltpu.sync_copy(den_sh.at[pl.ds(nb + k * _DZR, _DZR)], zd_v)
        pltpu.sync_copy(zd_v, den_out.at[pl.ds(c * _NP + nb + k * _DZR, _DZR)])


_edge_pass = pl.kernel(
    _edge_body,
    out_type=[jax.ShapeDtypeStruct((_NC, _NP, _D), _f32),
              jax.ShapeDtypeStruct((_NC * _NP,), _f32)],
    mesh=plsc.VectorSubcoreMesh(core_axis_name="c", subcore_axis_name="s"),
    compiler_params=pltpu.CompilerParams(
        needs_layout_passes=False, use_tc_tiling_on_sc=False),
    scratch_types=[
        pltpu.VMEM_SHARED((_NP, _D), _f32),
        pltpu.VMEM_SHARED((_NP,), _f32),
        [pltpu.VMEM((2, _CH), _i32) for _ in range(3)],
        [pltpu.VMEM((_CH,), _f32) for _ in range(3)],
        [pltpu.VMEM((_CH,), _f32) for _ in range(2)],
        pltpu.VMEM((_CH,), _f32),
        [pltpu.VMEM((_CH, _D), _f32) for _ in range(2)],
        [pltpu.VMEM((_CH, _D), _f32) for _ in range(2)],
        pltpu.VMEM((_CH, _D), _f32),
        pltpu.VMEM((_L,), _f32),
        pltpu.VMEM((_ZR, _D), _f32),
        pltpu.VMEM((_DZR,), _f32),
        [pltpu.SemaphoreType.DMA for _ in range(3)],
        [pltpu.SemaphoreType.DMA for _ in range(2)],
    ],
)


def _gather_body(x2, hidx, aidx, xh_out, xa_out, idx_v, rows_v, sem):
    c = lax.axis_index("c")
    s = lax.axis_index("s")
    wid = s * _NC + c
    bpw = _B // _NW
    nj = bpw // _CH
    base = wid * bpw
    for idx_hbm, out_hbm in ((hidx, xh_out), (aidx, xa_out)):
        for j in range(nj):
            pltpu.sync_copy(idx_hbm.at[pl.ds(base + j * _CH, _CH)], idx_v.at[j])
        for j in range(nj):
            pltpu.async_copy(x2.at[idx_v.at[j]],
                             rows_v.at[pl.ds(j * _CH, _CH)], sem).wait()
        pltpu.sync_copy(rows_v, out_hbm.at[pl.ds(base, bpw)])


_head_gather = pl.kernel(
    _gather_body,
    out_type=[jax.ShapeDtypeStruct((_B, _D), _f32),
              jax.ShapeDtypeStruct((_B, _D), _f32)],
    mesh=plsc.VectorSubcoreMesh(core_axis_name="c", subcore_axis_name="s"),
    compiler_params=pltpu.CompilerParams(
        needs_layout_passes=False, use_tc_tiling_on_sc=False),
    scratch_types=[
        pltpu.VMEM((_B // _NW // _CH, _CH), _i32),
        pltpu.VMEM((_B // _NW, _D), _f32),
        pltpu.SemaphoreType.DMA,
    ],
)


_TCR = 2048
_TCG = _NP // _TCR


def _prep_body(x_ref, xn_ref, nrm_ref):
    x = x_ref[...]
    n = jnp.sqrt(jnp.sum(x * x, axis=1)) + 1e-12
    xn_ref[...] = x / n[:, None]
    nrm_ref[...] = n


_prep = pl.pallas_call(
    _prep_body,
    grid=(_TCG,),
    in_specs=[pl.BlockSpec((_TCR, _D), lambda i: (i, 0))],
    out_specs=[pl.BlockSpec((_TCR, _D), lambda i: (i, 0)),
               pl.BlockSpec((_TCR,), lambda i: (i,))],
    out_shape=[jax.ShapeDtypeStruct((_NP, _D), _f32),
               jax.ShapeDtypeStruct((_NP,), _f32)],
)


def _combine_body(acc_ref, den0_ref, den1_ref, x_ref, xn_ref, nrm_ref):
    a = acc_ref[0] + acc_ref[1]
    dsum = den0_ref[...] + den1_ref[...]
    x = a / (dsum[:, None] + 1e-16)
    x = jnp.where(x >= 0, x, 0.01 * x)
    x_ref[...] = x
    n = jnp.sqrt(jnp.sum(x * x, axis=1)) + 1e-12
    xn_ref[...] = x / n[:, None]
    nrm_ref[...] = n


_combine = pl.pallas_call(
    _combine_body,
    grid=(_TCG,),
    in_specs=[pl.BlockSpec((_NC, _TCR, _D), lambda i: (0, i, 0)),
              pl.BlockSpec((_TCR,), lambda i: (i,)),
              pl.BlockSpec((_TCR,), lambda i: (i + _TCG,))],
    out_specs=[pl.BlockSpec((_TCR, _D), lambda i: (i, 0)),
               pl.BlockSpec((_TCR, _D), lambda i: (i, 0)),
               pl.BlockSpec((_TCR,), lambda i: (i,))],
    out_shape=[jax.ShapeDtypeStruct((_NP, _D), _f32),
               jax.ShapeDtypeStruct((_NP, _D), _f32),
               jax.ShapeDtypeStruct((_NP,), _f32)],
)


def _mlp_body(xh_ref, xa_ref, w1_ref, c1_ref, w2_ref, c2_ref, w3_ref, c3_ref,
              out_ref):
    def lk(v):
        return jnp.where(v >= 0, v, 0.01 * v)

    def mm(u, wr, cr):
        return lk(jnp.dot(u, wr[...], preferred_element_type=_f32,
                          precision=lax.Precision.HIGHEST) + cr[...])

    h = jnp.concatenate([xh_ref[...], xa_ref[...]], axis=1)
    out_ref[...] = mm(mm(mm(h, w1_ref, c1_ref), w2_ref, c2_ref),
                      w3_ref, c3_ref)


_MBR = 2048

_mlp = pl.pallas_call(
    _mlp_body,
    grid=(_B // _MBR,),
    in_specs=[pl.BlockSpec((_MBR, _D), lambda i: (i, 0)),
              pl.BlockSpec((_MBR, _D), lambda i: (i, 0)),
              pl.BlockSpec((2 * _D, 64), lambda i: (0, 0)),
              pl.BlockSpec((64,), lambda i: (0,)),
              pl.BlockSpec((64, 32), lambda i: (0, 0)),
              pl.BlockSpec((32,), lambda i: (0,)),
              pl.BlockSpec((32, 128), lambda i: (0, 0)),
              pl.BlockSpec((128,), lambda i: (0,))],
    out_specs=pl.BlockSpec((_MBR, 128), lambda i: (i, 0)),
    out_shape=jax.ShapeDtypeStruct((_B, 128), _f32),
)


def _lsm_body(h_ref, out_ref):
    h = h_ref[...]
    m = jnp.max(h, axis=0, keepdims=True)
    lse = m + jnp.log(jnp.sum(jnp.exp(h - m), axis=0, keepdims=True))
    out_ref[...] = h - lse


_lsm = pl.pallas_call(
    _lsm_body,
    out_shape=jax.ShapeDtypeStruct((_B, 128), _f32),
)


@jax.jit
def kernel(edge_index, edge_weight, home, away, emb, beta1, beta2,
           W1, b1, W2, b2, W3, b3):
    bv1 = jnp.full((_L,), beta1, _f32)
    bv2 = jnp.full((_L,), beta2, _f32)

    xn0, nrm0 = _prep(jnp.pad(emb, ((0, _NP - _N), (0, 0))))
    acc1, den1 = _edge_pass(xn0, nrm0, edge_index, edge_weight, bv1)
    x1, xn1, nrm1 = _combine(acc1, den1, den1)
    del x1
    acc2, den2 = _edge_pass(xn1, nrm1, edge_index, edge_weight, bv2)
    x2, _, _ = _combine(acc2, den2, den2)
    xh, xa = _head_gather(x2, home, away)

    w3p = jnp.zeros((32, 128), _f32).at[:, :_TD].set(W3)
    b3p = jnp.zeros((128,), _f32).at[:_TD].set(b3)
    out = _lsm(_mlp(xh, xa, W1, b1, W2, b2, w3p, b3p))
    return out[:, :_TD]
